# Initial kernel scaffold; baseline (speedup 1.0000x reference)
import jax
import jax.numpy as jnp
from jax.experimental import pallas as pl


def kernel(in_feat, edge_index, edge_types, W1, loop1, b1, W2, loop2, b2):
    raise NotImplementedError("write your pallas kernel here")


# speedup vs baseline: 17.6819x; 17.6819x over previous
import functools

import jax
import jax.numpy as jnp
from jax import lax
from jax.experimental import pallas as pl
from jax.experimental.pallas import tpu as pltpu
from jax.experimental.pallas import tpu_sc as plsc

_N, _E, _D, _R = 10000, 320000, 128, 8
_NC, _NS = 2, 16
_NW = _NC * _NS
_K = 128
_EC = _E // _K
_BN = 1000
_NB = _N // _BN
_NPAD = 10240
_RT = _NPAD // _NS


def _mm_body(h_ref, w_ref, o_ref):
    o_ref[...] = jnp.dot(h_ref[...], w_ref[0], preferred_element_type=jnp.float32)


def _table(h, wcat):
    return pl.pallas_call(
        _mm_body,
        grid=(_NB, _R + 1),
        in_specs=[
            pl.BlockSpec((_BN, _D), lambda i, r: (i, 0)),
            pl.BlockSpec((1, _D, _D), lambda i, r: (r, 0, 0)),
        ],
        out_specs=pl.BlockSpec((_BN, _D), lambda i, r: (r * _NB + i, 0)),
        out_shape=jax.ShapeDtypeStruct(((_R + 1) * _N, _D), jnp.float32),
    )(h, wcat)


def _combine_mm_body(p_ref, sl_ref, b_ref, w_ref, o_ref):
    h = jnp.maximum(p_ref[0] + p_ref[1] + sl_ref[...] + b_ref[...], 0.0)
    o_ref[...] = jnp.dot(h, w_ref[0], preferred_element_type=jnp.float32)


def _combine_table(parts, table1, brow, wcat):
    return pl.pallas_call(
        _combine_mm_body,
        grid=(_NB, _R + 1),
        in_specs=[
            pl.BlockSpec((_NC, _BN, _D), lambda i, r: (0, i, 0)),
            pl.BlockSpec((_BN, _D), lambda i, r: (_R * _NB + i, 0)),
            pl.BlockSpec((1, _D), lambda i, r: (0, 0)),
            pl.BlockSpec((1, _D, _D), lambda i, r: (r, 0, 0)),
        ],
        out_specs=pl.BlockSpec((_BN, _D), lambda i, r: (r * _NB + i, 0)),
        out_shape=jax.ShapeDtypeStruct(((_R + 1) * _N, _D), jnp.float32),
    )(parts, table1, brow, wcat)


def _final_body(p_ref, sl_ref, b_ref, o_ref):
    h = p_ref[0] + p_ref[1] + sl_ref[...] + b_ref[...]
    nrm = jnp.sqrt(jnp.sum(h * h, axis=1, keepdims=True))
    o_ref[...] = h / jnp.maximum(nrm, 1e-12)


def _final(parts, table2, brow):
    return pl.pallas_call(
        _final_body,
        grid=(_NB,),
        in_specs=[
            pl.BlockSpec((_NC, _BN, _D), lambda i: (0, i, 0)),
            pl.BlockSpec((_BN, _D), lambda i: (_R * _NB + i, 0)),
            pl.BlockSpec((1, _D), lambda i: (0, 0)),
        ],
        out_specs=pl.BlockSpec((_BN, _D), lambda i: (i, 0)),
        out_shape=jax.ShapeDtypeStruct((_N, _D), jnp.float32),
    )(parts, table2, brow)


def _sc_scatter(table, gidx, dst, zeros):
    mesh = plsc.VectorSubcoreMesh(core_axis_name="c", subcore_axis_name="s")

    @functools.partial(
        pl.kernel,
        mesh=mesh,
        out_type=jax.ShapeDtypeStruct((_NC, _NPAD, _D), jnp.float32),
        scratch_types=[
            pltpu.VMEM((_K,), jnp.int32),
            pltpu.VMEM((_K,), jnp.int32),
            pltpu.VMEM((_K, _D), jnp.float32),
            pltpu.VMEM_SHARED((_NPAD, _D), jnp.float32),
            pltpu.SemaphoreType.DMA,
        ],
    )
    def body(table_hbm, gidx_hbm, dst_hbm, zeros_hbm, out_hbm,
             idx_v, dst_v, rows_v, acc_sh, sem):
        c = lax.axis_index("c")
        s = lax.axis_index("s")
        w = s * _NC + c
        pltpu.sync_copy(zeros_hbm.at[pl.ds(s * _RT, _RT)],
                        acc_sh.at[pl.ds(s * _RT, _RT)])
        plsc.subcore_barrier()

        jmax = (_EC - w + _NW - 1) // _NW

        def step(j, carry):
            base = (w + j * _NW) * _K
            pltpu.sync_copy(gidx_hbm.at[pl.ds(base, _K)], idx_v)
            pltpu.sync_copy(dst_hbm.at[pl.ds(base, _K)], dst_v)
            pltpu.async_copy(table_hbm.at[idx_v], rows_v, sem).wait()
            pltpu.sync_copy(rows_v, acc_sh.at[dst_v], add=True)
            return carry

        lax.fori_loop(0, jmax, step, 0)
        plsc.subcore_barrier()
        pltpu.sync_copy(acc_sh.at[pl.ds(s * _RT, _RT)],
                        out_hbm.at[c, pl.ds(s * _RT, _RT)])

    return body(table, gidx, dst, zeros)


def kernel(in_feat, edge_index, edge_types, W1, loop1, b1, W2, loop2, b2):
    src = edge_index[0].astype(jnp.int32)
    dst = edge_index[1].astype(jnp.int32)
    et = edge_types.astype(jnp.int32)
    gidx = et * _N + src
    zeros = jnp.zeros((_NPAD, _D), jnp.float32)
    wcat1 = jnp.concatenate([W1, loop1[None]], axis=0)
    wcat2 = jnp.concatenate([W2, loop2[None]], axis=0)
    b1r = b1.reshape(1, _D)
    b2r = b2.reshape(1, _D)

    t1 = _table(in_feat, wcat1)
    p1 = _sc_scatter(t1, gidx, dst, zeros)[:, :_N]
    t2 = _combine_table(p1, t1, b1r, wcat2)
    p2 = _sc_scatter(t2, gidx, dst, zeros)[:, :_N]
    return _final(p2, t2, b2r)
